# Initial kernel scaffold; baseline (speedup 1.0000x reference)
import jax
import jax.numpy as jnp
from jax.experimental import pallas as pl


def kernel(x, edge_index, batch_index, W0, b0, W1, b1, W2, b2, pool_w):
    raise NotImplementedError("write your pallas kernel here")


# speedup vs baseline: 11.3676x; 11.3676x over previous
import jax
import jax.numpy as jnp
from jax import lax
from jax.experimental import pallas as pl
from jax.experimental.pallas import tpu as pltpu
from jax.experimental.pallas import tpu_sc as plsc

N = 10000
E = 160000
NB = E // 128
D = 256
B = 64
HALF = 128
GMAX = 80
RA = 624

_mesh = plsc.VectorSubcoreMesh(core_axis_name="c", subcore_axis_name="s")


def _zero16():
    return jnp.zeros((16,), jnp.float32)


def _tile_batches(s):
    lo = s * 80
    cnt = jnp.where(s < 15, 80, 50)
    return lo, cnt


def _fill_const(buf, rows, vec):
    def body(t, carry):
        buf[t // 8, pl.ds((t % 8) * 16, 16)] = vec
        return carry
    lax.fori_loop(0, rows * 8, body, 0)


def _zero_acc(acc, zbuf, s):
    base = s * RA

    @pl.when(s < 15)
    def _():
        for k in range(4):
            pltpu.sync_copy(zbuf, acc.at[pl.ds(base + k * 128, 128)])
        pltpu.sync_copy(zbuf.at[pl.ds(0, RA - 512)],
                        acc.at[pl.ds(base + 512, RA - 512)])

    @pl.when(s == 15)
    def _():
        for k in range(5):
            pltpu.sync_copy(zbuf, acc.at[pl.ds(base + k * 128, 128)])


def _copy_out(acc, buf, out_half, s):
    base = s * RA

    @pl.when(s < 15)
    def _():
        for k in range(4):
            pltpu.sync_copy(acc.at[pl.ds(base + k * 128, 128)], buf)
            pltpu.sync_copy(buf, out_half.at[pl.ds(base + k * 128, 128)])
        pltpu.sync_copy(acc.at[pl.ds(base + 512, RA - 512)],
                        buf.at[pl.ds(0, RA - 512)])
        pltpu.sync_copy(buf.at[pl.ds(0, RA - 512)],
                        out_half.at[pl.ds(base + 512, RA - 512)])

    @pl.when(s == 15)
    def _():
        for k in range(5):
            pltpu.sync_copy(acc.at[pl.ds(base + k * 128, 128)], buf)
            pltpu.sync_copy(buf, out_half.at[pl.ds(base + k * 128, 128)])


def _sc_deg_body(er_hbm, ec_hbm, out_hbm, idx, ones, zbuf, acc):
    c = lax.axis_index("c")
    s = lax.axis_index("s")

    one16 = jnp.full((16,), 1.0, jnp.float32)
    _fill_const(ones, 128, one16)
    _fill_const(zbuf, 128, _zero16())
    _zero_acc(acc, zbuf, s)
    plsc.subcore_barrier()

    lo, cnt = _tile_batches(s)

    def body(g, carry):
        @pl.when(c == 0)
        def _():
            pltpu.sync_copy(er_hbm.at[pl.ds((lo + g) * 128, 128)], idx)

        @pl.when(c == 1)
        def _():
            pltpu.sync_copy(ec_hbm.at[pl.ds((lo + g) * 128, 128)], idx)

        pltpu.sync_copy(ones, acc.at[idx], add=True)
        return carry
    lax.fori_loop(0, cnt, body, 0)

    plsc.subcore_barrier()
    _copy_out(acc, zbuf, out_hbm.at[c], s)


_sc_deg = pl.kernel(
    _sc_deg_body,
    out_type=jax.ShapeDtypeStruct((2, N, HALF), jnp.float32),
    mesh=_mesh,
    scratch_types=[
        pltpu.VMEM((128,), jnp.int32),
        pltpu.VMEM((128, HALF), jnp.float32),
        pltpu.VMEM((128, HALF), jnp.float32),
        pltpu.VMEM_SHARED((N, HALF), jnp.float32),
    ],
)


def _sc_agg_body(lin_hbm, er_hbm, ec_hbm, out_hbm,
                 ridx, cidx, bufa, bufb, acc, sema, semb):
    c = lax.axis_index("c")
    s = lax.axis_index("s")

    _fill_const(bufa, 128, _zero16())
    _zero_acc(acc, bufa, s)
    plsc.subcore_barrier()

    lo, cnt = _tile_batches(s)
    table = lin_hbm.at[c]

    def body(g, carry):
        pltpu.sync_copy(er_hbm.at[pl.ds((lo + g) * 128, 128)], ridx)
        pltpu.sync_copy(ec_hbm.at[pl.ds((lo + g) * 128, 128)], cidx)
        ga = pltpu.async_copy(table.at[ridx], bufa, sema)
        gb = pltpu.async_copy(table.at[cidx], bufb, semb)
        ga.wait()
        pltpu.sync_copy(bufa, acc.at[cidx], add=True)
        gb.wait()
        pltpu.sync_copy(bufb, acc.at[ridx], add=True)
        return carry

    lax.fori_loop(0, cnt, body, 0)

    plsc.subcore_barrier()
    _copy_out(acc, bufa, out_hbm.at[c], s)


_sc_agg = pl.kernel(
    _sc_agg_body,
    out_type=jax.ShapeDtypeStruct((2, N, HALF), jnp.float32),
    mesh=_mesh,
    scratch_types=[
        pltpu.VMEM((128,), jnp.int32),
        pltpu.VMEM((128,), jnp.int32),
        pltpu.VMEM((128, HALF), jnp.float32),
        pltpu.VMEM((128, HALF), jnp.float32),
        pltpu.VMEM_SHARED((N, HALF), jnp.float32),
        pltpu.SemaphoreType.DMA,
        pltpu.SemaphoreType.DMA,
    ],
)


BM = 256
_GRID = (N + BM - 1) // BM


def _t0_body(x_ref, w_ref, b_ref, degp_ref, lin_ref, dis_ref):
    deg = degp_ref[0, :, 0:1] + degp_ref[1, :, 0:1] + 1.0
    dis = lax.rsqrt(deg)
    lin = lax.dot_general(x_ref[...], w_ref[...], (((1,), (1,)), ((), ())),
                          preferred_element_type=jnp.float32) + b_ref[...]
    linp = dis * lin
    dis_ref[...] = dis
    lin_ref[0] = linp[:, :HALF]
    lin_ref[1] = linp[:, HALF:]


def _tmid_body(agg_ref, lp_ref, dis_ref, w_ref, b_ref, out_ref):
    dis = dis_ref[...]
    pre = jnp.concatenate(
        [agg_ref[0] + lp_ref[0], agg_ref[1] + lp_ref[1]], axis=1)
    h = jnp.maximum(dis * pre, 0.0)
    lin = lax.dot_general(h, w_ref[...], (((1,), (1,)), ((), ())),
                          preferred_element_type=jnp.float32) + b_ref[...]
    linp = dis * lin
    out_ref[0] = linp[:, :HALF]
    out_ref[1] = linp[:, HALF:]


def _tfin_body(agg_ref, lp_ref, dis_ref, pw_ref, bi_ref, h_ref, z_ref):
    pre = jnp.concatenate(
        [agg_ref[0] + lp_ref[0], agg_ref[1] + lp_ref[1]], axis=1)
    h = jnp.maximum(dis_ref[...] * pre, 0.0)
    h_ref[...] = h
    scores = lax.dot_general(h, pw_ref[...], (((1,), (0,)), ((), ())),
                             preferred_element_type=jnp.float32)
    smax = jnp.max(scores)
    e = jnp.exp(scores - smax)
    sm = e / jnp.sum(e)
    oh = (bi_ref[...] == lax.broadcasted_iota(jnp.int32, (N, B), 1)
          ).astype(jnp.float32)
    z_ref[...] = lax.dot_general(oh, h * sm, (((0,), (0,)), ((), ())),
                                 preferred_element_type=jnp.float32)


_t0 = pl.pallas_call(
    _t0_body,
    grid=(_GRID,),
    in_specs=[
        pl.BlockSpec((BM, D), lambda i: (i, 0)),
        pl.BlockSpec((D, D), lambda i: (0, 0)),
        pl.BlockSpec((1, D), lambda i: (0, 0)),
        pl.BlockSpec((2, BM, HALF), lambda i: (0, i, 0)),
    ],
    out_specs=[
        pl.BlockSpec((2, BM, HALF), lambda i: (0, i, 0)),
        pl.BlockSpec((BM, 1), lambda i: (i, 0)),
    ],
    out_shape=[
        jax.ShapeDtypeStruct((2, N, HALF), jnp.float32),
        jax.ShapeDtypeStruct((N, 1), jnp.float32),
    ],
)

_tmid = pl.pallas_call(
    _tmid_body,
    grid=(_GRID,),
    in_specs=[
        pl.BlockSpec((2, BM, HALF), lambda i: (0, i, 0)),
        pl.BlockSpec((2, BM, HALF), lambda i: (0, i, 0)),
        pl.BlockSpec((BM, 1), lambda i: (i, 0)),
        pl.BlockSpec((D, D), lambda i: (0, 0)),
        pl.BlockSpec((1, D), lambda i: (0, 0)),
    ],
    out_specs=pl.BlockSpec((2, BM, HALF), lambda i: (0, i, 0)),
    out_shape=jax.ShapeDtypeStruct((2, N, HALF), jnp.float32),
)

_tfin = pl.pallas_call(
    _tfin_body,
    out_shape=[
        jax.ShapeDtypeStruct((N, D), jnp.float32),
        jax.ShapeDtypeStruct((B, D), jnp.float32),
    ],
)


def kernel(x, edge_index, batch_index, W0, b0, W1, b1, W2, b2, pool_w):
    er = edge_index[0]
    ec = edge_index[1]
    bi = batch_index.reshape(N, 1)

    degp = _sc_deg(er, ec)
    lin0, dis = _t0(x, W0, b0.reshape(1, D), degp)
    agg0 = _sc_agg(lin0, er, ec)
    lin1 = _tmid(agg0, lin0, dis, W1, b1.reshape(1, D))
    agg1 = _sc_agg(lin1, er, ec)
    lin2 = _tmid(agg1, lin1, dis, W2, b2.reshape(1, D))
    agg2 = _sc_agg(lin2, er, ec)
    h, z = _tfin(agg2, lin2, dis, pool_w.reshape(D, 1), bi)
    return (h, z)


# speedup vs baseline: 13.6180x; 1.1980x over previous
import jax
import jax.numpy as jnp
from jax import lax
from jax.experimental import pallas as pl
from jax.experimental.pallas import tpu as pltpu
from jax.experimental.pallas import tpu_sc as plsc

N = 10000
E = 160000
NB = E // 128
D = 256
B = 64
HALF = 128
GMAX = 80
RA = 624

_mesh = plsc.VectorSubcoreMesh(core_axis_name="c", subcore_axis_name="s")


def _zero16():
    return jnp.zeros((16,), jnp.float32)


def _tile_batches(s):
    lo = s * 80
    cnt = jnp.where(s < 15, 80, 50)
    return lo, cnt


def _fill_const(buf, rows, vec):
    def body(t, carry):
        buf[t // 8, pl.ds((t % 8) * 16, 16)] = vec
        return carry
    lax.fori_loop(0, rows * 8, body, 0)


def _zero_acc(acc, zbuf, s):
    base = s * RA

    @pl.when(s < 15)
    def _():
        for k in range(4):
            pltpu.sync_copy(zbuf, acc.at[pl.ds(base + k * 128, 128)])
        pltpu.sync_copy(zbuf.at[pl.ds(0, RA - 512)],
                        acc.at[pl.ds(base + 512, RA - 512)])

    @pl.when(s == 15)
    def _():
        for k in range(5):
            pltpu.sync_copy(zbuf, acc.at[pl.ds(base + k * 128, 128)])


def _copy_out(acc, buf, out_half, s):
    base = s * RA

    @pl.when(s < 15)
    def _():
        for k in range(4):
            pltpu.sync_copy(acc.at[pl.ds(base + k * 128, 128)], buf)
            pltpu.sync_copy(buf, out_half.at[pl.ds(base + k * 128, 128)])
        pltpu.sync_copy(acc.at[pl.ds(base + 512, RA - 512)],
                        buf.at[pl.ds(0, RA - 512)])
        pltpu.sync_copy(buf.at[pl.ds(0, RA - 512)],
                        out_half.at[pl.ds(base + 512, RA - 512)])

    @pl.when(s == 15)
    def _():
        for k in range(5):
            pltpu.sync_copy(acc.at[pl.ds(base + k * 128, 128)], buf)
            pltpu.sync_copy(buf, out_half.at[pl.ds(base + k * 128, 128)])


def _sc_deg_body(er_hbm, ec_hbm, out_hbm, idx, ones, zbuf, acc):
    c = lax.axis_index("c")
    s = lax.axis_index("s")

    one16 = jnp.full((16,), 1.0, jnp.float32)
    _fill_const(ones, 128, one16)
    _fill_const(zbuf, 128, _zero16())
    _zero_acc(acc, zbuf, s)
    plsc.subcore_barrier()

    lo, cnt = _tile_batches(s)

    def body(g, carry):
        @pl.when(c == 0)
        def _():
            pltpu.sync_copy(er_hbm.at[pl.ds((lo + g) * 128, 128)], idx)

        @pl.when(c == 1)
        def _():
            pltpu.sync_copy(ec_hbm.at[pl.ds((lo + g) * 128, 128)], idx)

        pltpu.sync_copy(ones, acc.at[idx], add=True)
        return carry
    lax.fori_loop(0, cnt, body, 0)

    plsc.subcore_barrier()
    _copy_out(acc, zbuf, out_hbm.at[c], s)


_sc_deg = pl.kernel(
    _sc_deg_body,
    out_type=jax.ShapeDtypeStruct((2, N, HALF), jnp.float32),
    mesh=_mesh,
    scratch_types=[
        pltpu.VMEM((128,), jnp.int32),
        pltpu.VMEM((128, HALF), jnp.float32),
        pltpu.VMEM((128, HALF), jnp.float32),
        pltpu.VMEM_SHARED((N, HALF), jnp.float32),
    ],
)


def _sc_agg_body(lin_hbm, er_hbm, ec_hbm, out_hbm,
                 ridx0, ridx1, cidx0, cidx1, bufa, bufb, acc,
                 sema, semb, isem0, isem1):
    c = lax.axis_index("c")
    s = lax.axis_index("s")

    _fill_const(bufa, 128, _zero16())
    _zero_acc(acc, bufa, s)
    plsc.subcore_barrier()

    lo, cnt = _tile_batches(s)
    table = lin_hbm.at[c]
    rset = (ridx0, ridx1)
    cset = (cidx0, cidx1)
    isem = (isem0, isem1)

    def _idx_start(g, p):
        a = pltpu.async_copy(er_hbm.at[pl.ds((lo + g) * 128, 128)],
                             rset[p], isem[p])
        b = pltpu.async_copy(ec_hbm.at[pl.ds((lo + g) * 128, 128)],
                             cset[p], isem[p])
        return a, b

    def _idx_wait(g, p):
        pltpu.make_async_copy(er_hbm.at[pl.ds((lo + g) * 128, 128)],
                              rset[p], isem[p]).wait()
        pltpu.make_async_copy(ec_hbm.at[pl.ds((lo + g) * 128, 128)],
                              cset[p], isem[p]).wait()

    def _streams(p):
        ga = pltpu.async_copy(table.at[rset[p]], bufa, sema)
        gb = pltpu.async_copy(table.at[cset[p]], bufb, semb)
        ga.wait()
        pltpu.sync_copy(bufa, acc.at[cset[p]], add=True)
        gb.wait()
        pltpu.sync_copy(bufb, acc.at[rset[p]], add=True)

    _idx_start(0, 0)
    cnt2 = cnt // 2

    def body(k, carry):
        g = 2 * k
        _idx_wait(g, 0)
        _idx_start(g + 1, 1)
        _streams(0)

        _idx_wait(g + 1, 1)

        @pl.when(k < cnt2 - 1)
        def _():
            _idx_start(g + 2, 0)

        _streams(1)
        return carry

    lax.fori_loop(0, cnt2, body, 0)

    plsc.subcore_barrier()
    _copy_out(acc, bufa, out_hbm.at[c], s)


_sc_agg = pl.kernel(
    _sc_agg_body,
    out_type=jax.ShapeDtypeStruct((2, N, HALF), jnp.float32),
    mesh=_mesh,
    scratch_types=[
        pltpu.VMEM((128,), jnp.int32),
        pltpu.VMEM((128,), jnp.int32),
        pltpu.VMEM((128,), jnp.int32),
        pltpu.VMEM((128,), jnp.int32),
        pltpu.VMEM((128, HALF), jnp.float32),
        pltpu.VMEM((128, HALF), jnp.float32),
        pltpu.VMEM_SHARED((N, HALF), jnp.float32),
        pltpu.SemaphoreType.DMA,
        pltpu.SemaphoreType.DMA,
        pltpu.SemaphoreType.DMA,
        pltpu.SemaphoreType.DMA,
    ],
)


BM = 256
_GRID = (N + BM - 1) // BM


def _t0_body(x_ref, w_ref, b_ref, degp_ref, lin_ref, dis_ref):
    deg = degp_ref[0, :, 0:1] + degp_ref[1, :, 0:1] + 1.0
    dis = lax.rsqrt(deg)
    lin = lax.dot_general(x_ref[...], w_ref[...], (((1,), (1,)), ((), ())),
                          preferred_element_type=jnp.float32) + b_ref[...]
    linp = dis * lin
    dis_ref[...] = dis
    lin_ref[0] = linp[:, :HALF]
    lin_ref[1] = linp[:, HALF:]


def _tmid_body(agg_ref, lp_ref, dis_ref, w_ref, b_ref, out_ref):
    dis = dis_ref[...]
    pre = jnp.concatenate(
        [agg_ref[0] + lp_ref[0], agg_ref[1] + lp_ref[1]], axis=1)
    h = jnp.maximum(dis * pre, 0.0)
    lin = lax.dot_general(h, w_ref[...], (((1,), (1,)), ((), ())),
                          preferred_element_type=jnp.float32) + b_ref[...]
    linp = dis * lin
    out_ref[0] = linp[:, :HALF]
    out_ref[1] = linp[:, HALF:]


def _tfin_body(agg_ref, lp_ref, dis_ref, pw_ref, bi_ref, h_ref, z_ref):
    pre = jnp.concatenate(
        [agg_ref[0] + lp_ref[0], agg_ref[1] + lp_ref[1]], axis=1)
    h = jnp.maximum(dis_ref[...] * pre, 0.0)
    h_ref[...] = h
    scores = lax.dot_general(h, pw_ref[...], (((1,), (0,)), ((), ())),
                             preferred_element_type=jnp.float32)
    smax = jnp.max(scores)
    e = jnp.exp(scores - smax)
    sm = e / jnp.sum(e)
    oh = (bi_ref[...] == lax.broadcasted_iota(jnp.int32, (N, B), 1)
          ).astype(jnp.float32)
    z_ref[...] = lax.dot_general(oh, h * sm, (((0,), (0,)), ((), ())),
                                 preferred_element_type=jnp.float32)


_t0 = pl.pallas_call(
    _t0_body,
    grid=(_GRID,),
    in_specs=[
        pl.BlockSpec((BM, D), lambda i: (i, 0)),
        pl.BlockSpec((D, D), lambda i: (0, 0)),
        pl.BlockSpec((1, D), lambda i: (0, 0)),
        pl.BlockSpec((2, BM, HALF), lambda i: (0, i, 0)),
    ],
    out_specs=[
        pl.BlockSpec((2, BM, HALF), lambda i: (0, i, 0)),
        pl.BlockSpec((BM, 1), lambda i: (i, 0)),
    ],
    out_shape=[
        jax.ShapeDtypeStruct((2, N, HALF), jnp.float32),
        jax.ShapeDtypeStruct((N, 1), jnp.float32),
    ],
)

_tmid = pl.pallas_call(
    _tmid_body,
    grid=(_GRID,),
    in_specs=[
        pl.BlockSpec((2, BM, HALF), lambda i: (0, i, 0)),
        pl.BlockSpec((2, BM, HALF), lambda i: (0, i, 0)),
        pl.BlockSpec((BM, 1), lambda i: (i, 0)),
        pl.BlockSpec((D, D), lambda i: (0, 0)),
        pl.BlockSpec((1, D), lambda i: (0, 0)),
    ],
    out_specs=pl.BlockSpec((2, BM, HALF), lambda i: (0, i, 0)),
    out_shape=jax.ShapeDtypeStruct((2, N, HALF), jnp.float32),
)

_tfin = pl.pallas_call(
    _tfin_body,
    out_shape=[
        jax.ShapeDtypeStruct((N, D), jnp.float32),
        jax.ShapeDtypeStruct((B, D), jnp.float32),
    ],
)


def kernel(x, edge_index, batch_index, W0, b0, W1, b1, W2, b2, pool_w):
    er = edge_index[0]
    ec = edge_index[1]
    bi = batch_index.reshape(N, 1)

    degp = _sc_deg(er, ec)
    lin0, dis = _t0(x, W0, b0.reshape(1, D), degp)
    agg0 = _sc_agg(lin0, er, ec)
    lin1 = _tmid(agg0, lin0, dis, W1, b1.reshape(1, D))
    agg1 = _sc_agg(lin1, er, ec)
    lin2 = _tmid(agg1, lin1, dis, W2, b2.reshape(1, D))
    agg2 = _sc_agg(lin2, er, ec)
    h, z = _tfin(agg2, lin2, dis, pool_w.reshape(D, 1), bi)
    return (h, z)


# speedup vs baseline: 15.3494x; 1.1271x over previous
import jax
import jax.numpy as jnp
from jax import lax
from jax.experimental import pallas as pl
from jax.experimental.pallas import tpu as pltpu
from jax.experimental.pallas import tpu_sc as plsc

N = 10000
E = 160000
NB = E // 128
D = 256
B = 64
HALF = 128
GMAX = 80
RA = 624

_mesh = plsc.VectorSubcoreMesh(core_axis_name="c", subcore_axis_name="s")


def _zero16():
    return jnp.zeros((16,), jnp.float32)


def _tile_batches(s):
    lo = s * 80
    cnt = jnp.where(s < 15, 80, 50)
    return lo, cnt


def _fill_const(buf, rows, vec):
    def body(t, carry):
        buf[t // 8, pl.ds((t % 8) * 16, 16)] = vec
        return carry
    lax.fori_loop(0, rows * 8, body, 0)


def _zero_acc(acc, zbuf, s):
    base = s * RA

    @pl.when(s < 15)
    def _():
        for k in range(4):
            pltpu.sync_copy(zbuf, acc.at[pl.ds(base + k * 128, 128)])
        pltpu.sync_copy(zbuf.at[pl.ds(0, RA - 512)],
                        acc.at[pl.ds(base + 512, RA - 512)])

    @pl.when(s == 15)
    def _():
        for k in range(5):
            pltpu.sync_copy(zbuf, acc.at[pl.ds(base + k * 128, 128)])


def _copy_out(acc, buf, out_half, s):
    base = s * RA

    @pl.when(s < 15)
    def _():
        for k in range(4):
            pltpu.sync_copy(acc.at[pl.ds(base + k * 128, 128)], buf)
            pltpu.sync_copy(buf, out_half.at[pl.ds(base + k * 128, 128)])
        pltpu.sync_copy(acc.at[pl.ds(base + 512, RA - 512)],
                        buf.at[pl.ds(0, RA - 512)])
        pltpu.sync_copy(buf.at[pl.ds(0, RA - 512)],
                        out_half.at[pl.ds(base + 512, RA - 512)])

    @pl.when(s == 15)
    def _():
        for k in range(5):
            pltpu.sync_copy(acc.at[pl.ds(base + k * 128, 128)], buf)
            pltpu.sync_copy(buf, out_half.at[pl.ds(base + k * 128, 128)])


def _sc_deg_body(er_hbm, ec_hbm, out_hbm, idx, ones, zbuf, acc):
    c = lax.axis_index("c")
    s = lax.axis_index("s")

    one16 = jnp.full((16,), 1.0, jnp.float32)
    _fill_const(ones, 128, one16)
    _fill_const(zbuf, 128, _zero16())
    _zero_acc(acc, zbuf, s)
    plsc.subcore_barrier()

    lo, cnt = _tile_batches(s)

    def body(g, carry):
        @pl.when(c == 0)
        def _():
            pltpu.sync_copy(er_hbm.at[pl.ds((lo + g) * 128, 128)], idx)

        @pl.when(c == 1)
        def _():
            pltpu.sync_copy(ec_hbm.at[pl.ds((lo + g) * 128, 128)], idx)

        pltpu.sync_copy(ones, acc.at[idx], add=True)
        return carry
    lax.fori_loop(0, cnt, body, 0)

    plsc.subcore_barrier()
    _copy_out(acc, zbuf, out_hbm.at[c], s)


_sc_deg = pl.kernel(
    _sc_deg_body,
    out_type=jax.ShapeDtypeStruct((2, N, HALF), jnp.float32),
    mesh=_mesh,
    scratch_types=[
        pltpu.VMEM((128,), jnp.int32),
        pltpu.VMEM((128, HALF), jnp.float32),
        pltpu.VMEM((128, HALF), jnp.float32),
        pltpu.VMEM_SHARED((N, HALF), jnp.float32),
    ],
)


def _sc_agg_body(lin_hbm, er_hbm, ec_hbm, out_hbm,
                 ridx0, ridx1, cidx0, cidx1, bufa, bufb, acc,
                 sema, semb, isem0, isem1, ssa, ssb):
    c = lax.axis_index("c")
    s = lax.axis_index("s")

    _fill_const(bufa, 128, _zero16())
    _zero_acc(acc, bufa, s)
    plsc.subcore_barrier()

    lo, cnt = _tile_batches(s)
    table = lin_hbm.at[c]
    cnt2 = cnt // 2

    def _idx_start(g, rdst, cdst, sem):
        pltpu.async_copy(er_hbm.at[pl.ds((lo + g) * 128, 128)], rdst, sem)
        pltpu.async_copy(ec_hbm.at[pl.ds((lo + g) * 128, 128)], cdst, sem)

    def _idx_wait(g, rdst, cdst, sem):
        pltpu.make_async_copy(er_hbm.at[pl.ds((lo + g) * 128, 128)],
                              rdst, sem).wait()
        pltpu.make_async_copy(ec_hbm.at[pl.ds((lo + g) * 128, 128)],
                              cdst, sem).wait()

    pltpu.sync_copy(er_hbm.at[pl.ds(lo * 128, 128)], ridx0)
    pltpu.sync_copy(ec_hbm.at[pl.ds(lo * 128, 128)], cidx0)
    pltpu.async_copy(table.at[ridx0], bufa, sema)

    def body(k, carry):
        g = 2 * k
        pltpu.make_async_copy(table.at[ridx0], bufa, sema).wait()
        pltpu.async_copy(bufa, acc.at[cidx0], ssa, add=True)

        @pl.when(k > 0)
        def _():
            pltpu.make_async_copy(bufb, acc.at[ridx1], ssb).wait()
        _idx_start(g + 1, ridx1, cidx1, isem1)
        pltpu.async_copy(table.at[cidx0], bufb, semb)

        pltpu.make_async_copy(table.at[cidx0], bufb, semb).wait()
        pltpu.async_copy(bufb, acc.at[ridx0], ssb, add=True)
        pltpu.make_async_copy(bufa, acc.at[cidx0], ssa).wait()
        _idx_wait(g + 1, ridx1, cidx1, isem1)
        pltpu.async_copy(table.at[ridx1], bufa, sema)

        pltpu.make_async_copy(table.at[ridx1], bufa, sema).wait()
        pltpu.async_copy(bufa, acc.at[cidx1], ssa, add=True)
        pltpu.make_async_copy(bufb, acc.at[ridx0], ssb).wait()

        @pl.when(k < cnt2 - 1)
        def _():
            _idx_start(g + 2, ridx0, cidx0, isem0)
        pltpu.async_copy(table.at[cidx1], bufb, semb)

        pltpu.make_async_copy(table.at[cidx1], bufb, semb).wait()
        pltpu.async_copy(bufb, acc.at[ridx1], ssb, add=True)
        pltpu.make_async_copy(bufa, acc.at[cidx1], ssa).wait()

        @pl.when(k < cnt2 - 1)
        def _():
            _idx_wait(g + 2, ridx0, cidx0, isem0)
            pltpu.async_copy(table.at[ridx0], bufa, sema)

        return carry

    lax.fori_loop(0, cnt2, body, 0)
    pltpu.make_async_copy(bufb, acc.at[ridx1], ssb).wait()

    plsc.subcore_barrier()
    _copy_out(acc, bufa, out_hbm.at[c], s)


_sc_agg = pl.kernel(
    _sc_agg_body,
    out_type=jax.ShapeDtypeStruct((2, N, HALF), jnp.float32),
    mesh=_mesh,
    scratch_types=[
        pltpu.VMEM((128,), jnp.int32),
        pltpu.VMEM((128,), jnp.int32),
        pltpu.VMEM((128,), jnp.int32),
        pltpu.VMEM((128,), jnp.int32),
        pltpu.VMEM((128, HALF), jnp.float32),
        pltpu.VMEM((128, HALF), jnp.float32),
        pltpu.VMEM_SHARED((N, HALF), jnp.float32),
        pltpu.SemaphoreType.DMA,
        pltpu.SemaphoreType.DMA,
        pltpu.SemaphoreType.DMA,
        pltpu.SemaphoreType.DMA,
        pltpu.SemaphoreType.DMA,
        pltpu.SemaphoreType.DMA,
    ],
)


BM = 256
_GRID = (N + BM - 1) // BM


def _t0_body(x_ref, w_ref, b_ref, degp_ref, lin_ref, dis_ref):
    deg = degp_ref[0, :, 0:1] + degp_ref[1, :, 0:1] + 1.0
    dis = lax.rsqrt(deg)
    lin = lax.dot_general(x_ref[...], w_ref[...], (((1,), (1,)), ((), ())),
                          preferred_element_type=jnp.float32) + b_ref[...]
    linp = dis * lin
    dis_ref[...] = dis
    lin_ref[0] = linp[:, :HALF]
    lin_ref[1] = linp[:, HALF:]


def _tmid_body(agg_ref, lp_ref, dis_ref, w_ref, b_ref, out_ref):
    dis = dis_ref[...]
    pre = jnp.concatenate(
        [agg_ref[0] + lp_ref[0], agg_ref[1] + lp_ref[1]], axis=1)
    h = jnp.maximum(dis * pre, 0.0)
    lin = lax.dot_general(h, w_ref[...], (((1,), (1,)), ((), ())),
                          preferred_element_type=jnp.float32) + b_ref[...]
    linp = dis * lin
    out_ref[0] = linp[:, :HALF]
    out_ref[1] = linp[:, HALF:]


def _tfin_body(agg_ref, lp_ref, dis_ref, pw_ref, bi_ref, h_ref, z_ref):
    pre = jnp.concatenate(
        [agg_ref[0] + lp_ref[0], agg_ref[1] + lp_ref[1]], axis=1)
    h = jnp.maximum(dis_ref[...] * pre, 0.0)
    h_ref[...] = h
    scores = lax.dot_general(h, pw_ref[...], (((1,), (0,)), ((), ())),
                             preferred_element_type=jnp.float32)
    smax = jnp.max(scores)
    e = jnp.exp(scores - smax)
    sm = e / jnp.sum(e)
    oh = (bi_ref[...] == lax.broadcasted_iota(jnp.int32, (N, B), 1)
          ).astype(jnp.float32)
    z_ref[...] = lax.dot_general(oh, h * sm, (((0,), (0,)), ((), ())),
                                 preferred_element_type=jnp.float32)


_t0 = pl.pallas_call(
    _t0_body,
    grid=(_GRID,),
    in_specs=[
        pl.BlockSpec((BM, D), lambda i: (i, 0)),
        pl.BlockSpec((D, D), lambda i: (0, 0)),
        pl.BlockSpec((1, D), lambda i: (0, 0)),
        pl.BlockSpec((2, BM, HALF), lambda i: (0, i, 0)),
    ],
    out_specs=[
        pl.BlockSpec((2, BM, HALF), lambda i: (0, i, 0)),
        pl.BlockSpec((BM, 1), lambda i: (i, 0)),
    ],
    out_shape=[
        jax.ShapeDtypeStruct((2, N, HALF), jnp.float32),
        jax.ShapeDtypeStruct((N, 1), jnp.float32),
    ],
)

_tmid = pl.pallas_call(
    _tmid_body,
    grid=(_GRID,),
    in_specs=[
        pl.BlockSpec((2, BM, HALF), lambda i: (0, i, 0)),
        pl.BlockSpec((2, BM, HALF), lambda i: (0, i, 0)),
        pl.BlockSpec((BM, 1), lambda i: (i, 0)),
        pl.BlockSpec((D, D), lambda i: (0, 0)),
        pl.BlockSpec((1, D), lambda i: (0, 0)),
    ],
    out_specs=pl.BlockSpec((2, BM, HALF), lambda i: (0, i, 0)),
    out_shape=jax.ShapeDtypeStruct((2, N, HALF), jnp.float32),
)

_tfin = pl.pallas_call(
    _tfin_body,
    out_shape=[
        jax.ShapeDtypeStruct((N, D), jnp.float32),
        jax.ShapeDtypeStruct((B, D), jnp.float32),
    ],
)


def kernel(x, edge_index, batch_index, W0, b0, W1, b1, W2, b2, pool_w):
    er = edge_index[0]
    ec = edge_index[1]
    bi = batch_index.reshape(N, 1)

    degp = _sc_deg(er, ec)
    lin0, dis = _t0(x, W0, b0.reshape(1, D), degp)
    agg0 = _sc_agg(lin0, er, ec)
    lin1 = _tmid(agg0, lin0, dis, W1, b1.reshape(1, D))
    agg1 = _sc_agg(lin1, er, ec)
    lin2 = _tmid(agg1, lin1, dis, W2, b2.reshape(1, D))
    agg2 = _sc_agg(lin2, er, ec)
    h, z = _tfin(agg2, lin2, dis, pool_w.reshape(D, 1), bi)
    return (h, z)


# speedup vs baseline: 15.9168x; 1.0370x over previous
import jax
import jax.numpy as jnp
from jax import lax
from jax.experimental import pallas as pl
from jax.experimental.pallas import tpu as pltpu
from jax.experimental.pallas import tpu_sc as plsc

N = 10000
E = 160000
NB = E // 128
D = 256
B = 64
HALF = 128
GMAX = 80
RA = 624

_mesh = plsc.VectorSubcoreMesh(core_axis_name="c", subcore_axis_name="s")


def _zero16():
    return jnp.zeros((16,), jnp.float32)


def _tile_batches(s):
    lo = s * 80
    cnt = jnp.where(s < 15, 80, 50)
    return lo, cnt


def _fill_const(buf, rows, vec):
    def body(t, carry):
        buf[t // 8, pl.ds((t % 8) * 16, 16)] = vec
        return carry
    lax.fori_loop(0, rows * 8, body, 0)


def _zero_acc(acc, zbuf, s):
    base = s * RA

    @pl.when(s < 15)
    def _():
        for k in range(4):
            pltpu.sync_copy(zbuf, acc.at[pl.ds(base + k * 128, 128)])
        pltpu.sync_copy(zbuf.at[pl.ds(0, RA - 512)],
                        acc.at[pl.ds(base + 512, RA - 512)])

    @pl.when(s == 15)
    def _():
        for k in range(5):
            pltpu.sync_copy(zbuf, acc.at[pl.ds(base + k * 128, 128)])


def _copy_out(acc, buf, out_half, s):
    base = s * RA

    @pl.when(s < 15)
    def _():
        for k in range(4):
            pltpu.sync_copy(acc.at[pl.ds(base + k * 128, 128)], buf)
            pltpu.sync_copy(buf, out_half.at[pl.ds(base + k * 128, 128)])
        pltpu.sync_copy(acc.at[pl.ds(base + 512, RA - 512)],
                        buf.at[pl.ds(0, RA - 512)])
        pltpu.sync_copy(buf.at[pl.ds(0, RA - 512)],
                        out_half.at[pl.ds(base + 512, RA - 512)])

    @pl.when(s == 15)
    def _():
        for k in range(5):
            pltpu.sync_copy(acc.at[pl.ds(base + k * 128, 128)], buf)
            pltpu.sync_copy(buf, out_half.at[pl.ds(base + k * 128, 128)])


def _sc_deg_body(ep_hbm, out_hbm, idx0, idx1, ones, zbuf, acc,
                 ssa, ssb, isem0, isem1):
    c = lax.axis_index("c")
    s = lax.axis_index("s")

    one16 = jnp.full((16,), 1.0, jnp.float32)
    _fill_const(ones, 128, one16)
    _fill_const(zbuf, 128, _zero16())
    _zero_acc(acc, zbuf, s)
    plsc.subcore_barrier()

    lo, cnt = _tile_batches(s)
    base = (c * 1250 + lo) * 128
    cnt2 = cnt // 2

    pltpu.sync_copy(ep_hbm.at[pl.ds(base, 128)], idx0)

    def body(k, carry):
        off = base + 2 * k * 128

        @pl.when(k > 0)
        def _():
            pltpu.make_async_copy(ep_hbm.at[pl.ds(off, 128)], idx0,
                                  isem0).wait()
        pltpu.async_copy(ones, acc.at[idx0], ssa, add=True)

        @pl.when(k > 0)
        def _():
            pltpu.make_async_copy(ones, acc.at[idx1], ssb).wait()
        pltpu.async_copy(ep_hbm.at[pl.ds(off + 128, 128)], idx1, isem1)

        pltpu.make_async_copy(ep_hbm.at[pl.ds(off + 128, 128)], idx1,
                              isem1).wait()
        pltpu.async_copy(ones, acc.at[idx1], ssb, add=True)
        pltpu.make_async_copy(ones, acc.at[idx0], ssa).wait()

        @pl.when(k < cnt2 - 1)
        def _():
            pltpu.async_copy(ep_hbm.at[pl.ds(off + 256, 128)], idx0, isem0)

        return carry

    lax.fori_loop(0, cnt2, body, 0)
    pltpu.make_async_copy(ones, acc.at[idx1], ssb).wait()

    plsc.subcore_barrier()
    _copy_out(acc, zbuf, out_hbm.at[c], s)


_sc_deg = pl.kernel(
    _sc_deg_body,
    out_type=jax.ShapeDtypeStruct((2, N, HALF), jnp.float32),
    mesh=_mesh,
    scratch_types=[
        pltpu.VMEM((128,), jnp.int32),
        pltpu.VMEM((128,), jnp.int32),
        pltpu.VMEM((128, HALF), jnp.float32),
        pltpu.VMEM((128, HALF), jnp.float32),
        pltpu.VMEM_SHARED((N, HALF), jnp.float32),
        pltpu.SemaphoreType.DMA,
        pltpu.SemaphoreType.DMA,
        pltpu.SemaphoreType.DMA,
        pltpu.SemaphoreType.DMA,
    ],
)


def _sc_agg_body(lin_hbm, er_hbm, ec_hbm, out_hbm,
                 ridx0, ridx1, cidx0, cidx1, bufa, bufb, acc,
                 sema, semb, isem0, isem1, ssa, ssb):
    c = lax.axis_index("c")
    s = lax.axis_index("s")

    _fill_const(bufa, 128, _zero16())
    _zero_acc(acc, bufa, s)
    plsc.subcore_barrier()

    lo, cnt = _tile_batches(s)
    table = lin_hbm.at[c]
    cnt2 = cnt // 2

    def _idx_start(g, rdst, cdst, sem):
        pltpu.async_copy(er_hbm.at[pl.ds((lo + g) * 128, 128)], rdst, sem)
        pltpu.async_copy(ec_hbm.at[pl.ds((lo + g) * 128, 128)], cdst, sem)

    def _idx_wait(g, rdst, cdst, sem):
        pltpu.make_async_copy(er_hbm.at[pl.ds((lo + g) * 128, 128)],
                              rdst, sem).wait()
        pltpu.make_async_copy(ec_hbm.at[pl.ds((lo + g) * 128, 128)],
                              cdst, sem).wait()

    pltpu.sync_copy(er_hbm.at[pl.ds(lo * 128, 128)], ridx0)
    pltpu.sync_copy(ec_hbm.at[pl.ds(lo * 128, 128)], cidx0)
    pltpu.async_copy(table.at[ridx0], bufa, sema)

    def body(k, carry):
        g = 2 * k
        pltpu.make_async_copy(table.at[ridx0], bufa, sema).wait()
        pltpu.async_copy(bufa, acc.at[cidx0], ssa, add=True)

        @pl.when(k > 0)
        def _():
            pltpu.make_async_copy(bufb, acc.at[ridx1], ssb).wait()
        _idx_start(g + 1, ridx1, cidx1, isem1)
        pltpu.async_copy(table.at[cidx0], bufb, semb)

        pltpu.make_async_copy(table.at[cidx0], bufb, semb).wait()
        pltpu.async_copy(bufb, acc.at[ridx0], ssb, add=True)
        pltpu.make_async_copy(bufa, acc.at[cidx0], ssa).wait()
        _idx_wait(g + 1, ridx1, cidx1, isem1)
        pltpu.async_copy(table.at[ridx1], bufa, sema)

        pltpu.make_async_copy(table.at[ridx1], bufa, sema).wait()
        pltpu.async_copy(bufa, acc.at[cidx1], ssa, add=True)
        pltpu.make_async_copy(bufb, acc.at[ridx0], ssb).wait()

        @pl.when(k < cnt2 - 1)
        def _():
            _idx_start(g + 2, ridx0, cidx0, isem0)
        pltpu.async_copy(table.at[cidx1], bufb, semb)

        pltpu.make_async_copy(table.at[cidx1], bufb, semb).wait()
        pltpu.async_copy(bufb, acc.at[ridx1], ssb, add=True)
        pltpu.make_async_copy(bufa, acc.at[cidx1], ssa).wait()

        @pl.when(k < cnt2 - 1)
        def _():
            _idx_wait(g + 2, ridx0, cidx0, isem0)
            pltpu.async_copy(table.at[ridx0], bufa, sema)

        return carry

    lax.fori_loop(0, cnt2, body, 0)
    pltpu.make_async_copy(bufb, acc.at[ridx1], ssb).wait()

    plsc.subcore_barrier()
    _copy_out(acc, bufa, out_hbm.at[c], s)


_sc_agg = pl.kernel(
    _sc_agg_body,
    out_type=jax.ShapeDtypeStruct((2, N, HALF), jnp.float32),
    mesh=_mesh,
    scratch_types=[
        pltpu.VMEM((128,), jnp.int32),
        pltpu.VMEM((128,), jnp.int32),
        pltpu.VMEM((128,), jnp.int32),
        pltpu.VMEM((128,), jnp.int32),
        pltpu.VMEM((128, HALF), jnp.float32),
        pltpu.VMEM((128, HALF), jnp.float32),
        pltpu.VMEM_SHARED((N, HALF), jnp.float32),
        pltpu.SemaphoreType.DMA,
        pltpu.SemaphoreType.DMA,
        pltpu.SemaphoreType.DMA,
        pltpu.SemaphoreType.DMA,
        pltpu.SemaphoreType.DMA,
        pltpu.SemaphoreType.DMA,
    ],
)


BM = 256
_GRID = (N + BM - 1) // BM


def _t0_body(x_ref, w_ref, b_ref, degp_ref, lin_ref, dis_ref):
    deg = degp_ref[0, :, 0:1] + degp_ref[1, :, 0:1] + 1.0
    dis = lax.rsqrt(deg)
    lin = lax.dot_general(x_ref[...], w_ref[...], (((1,), (1,)), ((), ())),
                          preferred_element_type=jnp.float32) + b_ref[...]
    linp = dis * lin
    dis_ref[...] = dis
    lin_ref[0] = linp[:, :HALF]
    lin_ref[1] = linp[:, HALF:]


def _tmid_body(agg_ref, lp_ref, dis_ref, w_ref, b_ref, out_ref):
    dis = dis_ref[...]
    pre = jnp.concatenate(
        [agg_ref[0] + lp_ref[0], agg_ref[1] + lp_ref[1]], axis=1)
    h = jnp.maximum(dis * pre, 0.0)
    lin = lax.dot_general(h, w_ref[...], (((1,), (1,)), ((), ())),
                          preferred_element_type=jnp.float32) + b_ref[...]
    linp = dis * lin
    out_ref[0] = linp[:, :HALF]
    out_ref[1] = linp[:, HALF:]


def _tfin_body(agg_ref, lp_ref, dis_ref, pw_ref, bi_ref, h_ref, z_ref):
    pre = jnp.concatenate(
        [agg_ref[0] + lp_ref[0], agg_ref[1] + lp_ref[1]], axis=1)
    h = jnp.maximum(dis_ref[...] * pre, 0.0)
    h_ref[...] = h
    scores = lax.dot_general(h, pw_ref[...], (((1,), (0,)), ((), ())),
                             preferred_element_type=jnp.float32)
    smax = jnp.max(scores)
    e = jnp.exp(scores - smax)
    sm = e / jnp.sum(e)
    oh = (bi_ref[...] == lax.broadcasted_iota(jnp.int32, (N, B), 1)
          ).astype(jnp.float32)
    z_ref[...] = lax.dot_general(oh, h * sm, (((0,), (0,)), ((), ())),
                                 preferred_element_type=jnp.float32)


_t0 = pl.pallas_call(
    _t0_body,
    grid=(_GRID,),
    in_specs=[
        pl.BlockSpec((BM, D), lambda i: (i, 0)),
        pl.BlockSpec((D, D), lambda i: (0, 0)),
        pl.BlockSpec((1, D), lambda i: (0, 0)),
        pl.BlockSpec((2, BM, HALF), lambda i: (0, i, 0)),
    ],
    out_specs=[
        pl.BlockSpec((2, BM, HALF), lambda i: (0, i, 0)),
        pl.BlockSpec((BM, 1), lambda i: (i, 0)),
    ],
    out_shape=[
        jax.ShapeDtypeStruct((2, N, HALF), jnp.float32),
        jax.ShapeDtypeStruct((N, 1), jnp.float32),
    ],
)

_tmid = pl.pallas_call(
    _tmid_body,
    grid=(_GRID,),
    in_specs=[
        pl.BlockSpec((2, BM, HALF), lambda i: (0, i, 0)),
        pl.BlockSpec((2, BM, HALF), lambda i: (0, i, 0)),
        pl.BlockSpec((BM, 1), lambda i: (i, 0)),
        pl.BlockSpec((D, D), lambda i: (0, 0)),
        pl.BlockSpec((1, D), lambda i: (0, 0)),
    ],
    out_specs=pl.BlockSpec((2, BM, HALF), lambda i: (0, i, 0)),
    out_shape=jax.ShapeDtypeStruct((2, N, HALF), jnp.float32),
)

_tfin = pl.pallas_call(
    _tfin_body,
    out_shape=[
        jax.ShapeDtypeStruct((N, D), jnp.float32),
        jax.ShapeDtypeStruct((B, D), jnp.float32),
    ],
)


def kernel(x, edge_index, batch_index, W0, b0, W1, b1, W2, b2, pool_w):
    er = edge_index[0]
    ec = edge_index[1]
    bi = batch_index.reshape(N, 1)

    degp = _sc_deg(edge_index.reshape(2 * E))
    lin0, dis = _t0(x, W0, b0.reshape(1, D), degp)
    agg0 = _sc_agg(lin0, er, ec)
    lin1 = _tmid(agg0, lin0, dis, W1, b1.reshape(1, D))
    agg1 = _sc_agg(lin1, er, ec)
    lin2 = _tmid(agg1, lin1, dis, W2, b2.reshape(1, D))
    agg2 = _sc_agg(lin2, er, ec)
    h, z = _tfin(agg2, lin2, dis, pool_w.reshape(D, 1), bi)
    return (h, z)


# speedup vs baseline: 15.9381x; 1.0013x over previous
import jax
import jax.numpy as jnp
from jax import lax
from jax.experimental import pallas as pl
from jax.experimental.pallas import tpu as pltpu
from jax.experimental.pallas import tpu_sc as plsc

N = 10000
E = 160000
NB = E // 128
D = 256
B = 64
HALF = 128
GMAX = 80
RA = 624

_mesh = plsc.VectorSubcoreMesh(core_axis_name="c", subcore_axis_name="s")


def _zero16():
    return jnp.zeros((16,), jnp.float32)


def _tile_batches(s):
    lo = s * 80
    cnt = jnp.where(s < 15, 80, 50)
    return lo, cnt


def _fill_const(buf, rows, vec):
    def body(t, carry):
        buf[t // 8, pl.ds((t % 8) * 16, 16)] = vec
        return carry
    lax.fori_loop(0, rows * 8, body, 0)


def _zero_acc(acc, zbuf, s):
    base = s * RA

    @pl.when(s < 15)
    def _():
        for k in range(4):
            pltpu.sync_copy(zbuf, acc.at[pl.ds(base + k * 128, 128)])
        pltpu.sync_copy(zbuf.at[pl.ds(0, RA - 512)],
                        acc.at[pl.ds(base + 512, RA - 512)])

    @pl.when(s == 15)
    def _():
        for k in range(5):
            pltpu.sync_copy(zbuf, acc.at[pl.ds(base + k * 128, 128)])


def _copy_out(acc, buf, out_half, s):
    base = s * RA

    @pl.when(s < 15)
    def _():
        for k in range(4):
            pltpu.sync_copy(acc.at[pl.ds(base + k * 128, 128)], buf)
            pltpu.sync_copy(buf, out_half.at[pl.ds(base + k * 128, 128)])
        pltpu.sync_copy(acc.at[pl.ds(base + 512, RA - 512)],
                        buf.at[pl.ds(0, RA - 512)])
        pltpu.sync_copy(buf.at[pl.ds(0, RA - 512)],
                        out_half.at[pl.ds(base + 512, RA - 512)])

    @pl.when(s == 15)
    def _():
        for k in range(5):
            pltpu.sync_copy(acc.at[pl.ds(base + k * 128, 128)], buf)
            pltpu.sync_copy(buf, out_half.at[pl.ds(base + k * 128, 128)])


def _sc_deg_body(ep_hbm, out_hbm, idx0, idx1, ones, zbuf, acc,
                 ssa, ssb, isem0, isem1):
    c = lax.axis_index("c")
    s = lax.axis_index("s")

    one16 = jnp.full((16,), 1.0, jnp.float32)
    _fill_const(ones, 128, one16)
    _fill_const(zbuf, 128, _zero16())
    _zero_acc(acc, zbuf, s)
    plsc.subcore_barrier()

    lo, cnt = _tile_batches(s)
    base = (c * 1250 + lo) * 128
    cnt2 = cnt // 2

    pltpu.sync_copy(ep_hbm.at[pl.ds(base, 128)], idx0)

    def body(k, carry):
        off = base + 2 * k * 128

        @pl.when(k > 0)
        def _():
            pltpu.make_async_copy(ep_hbm.at[pl.ds(off, 128)], idx0,
                                  isem0).wait()
        pltpu.async_copy(ones, acc.at[idx0], ssa, add=True)

        @pl.when(k > 0)
        def _():
            pltpu.make_async_copy(ones, acc.at[idx1], ssb).wait()
        pltpu.async_copy(ep_hbm.at[pl.ds(off + 128, 128)], idx1, isem1)

        pltpu.make_async_copy(ep_hbm.at[pl.ds(off + 128, 128)], idx1,
                              isem1).wait()
        pltpu.async_copy(ones, acc.at[idx1], ssb, add=True)
        pltpu.make_async_copy(ones, acc.at[idx0], ssa).wait()

        @pl.when(k < cnt2 - 1)
        def _():
            pltpu.async_copy(ep_hbm.at[pl.ds(off + 256, 128)], idx0, isem0)

        return carry

    lax.fori_loop(0, cnt2, body, 0)
    pltpu.make_async_copy(ones, acc.at[idx1], ssb).wait()

    plsc.subcore_barrier()
    _copy_out(acc, zbuf, out_hbm.at[c], s)


_sc_deg = pl.kernel(
    _sc_deg_body,
    out_type=jax.ShapeDtypeStruct((2, N, HALF), jnp.float32),
    mesh=_mesh,
    scratch_types=[
        pltpu.VMEM((128,), jnp.int32),
        pltpu.VMEM((128,), jnp.int32),
        pltpu.VMEM((128, HALF), jnp.float32),
        pltpu.VMEM((128, HALF), jnp.float32),
        pltpu.VMEM_SHARED((N, HALF), jnp.float32),
        pltpu.SemaphoreType.DMA,
        pltpu.SemaphoreType.DMA,
        pltpu.SemaphoreType.DMA,
        pltpu.SemaphoreType.DMA,
    ],
)


def _sc_agg_body(lin_hbm, er_hbm, ec_hbm, out_hbm,
                 ridx0, ridx1, cidx0, cidx1, bufa, bufb, acc,
                 sema, semb, isem0, isem1, ssa, ssb):
    c = lax.axis_index("c")
    s = lax.axis_index("s")

    _fill_const(bufa, 128, _zero16())
    _zero_acc(acc, bufa, s)
    plsc.subcore_barrier()

    lo, cnt = _tile_batches(s)
    table = lin_hbm.at[c]
    cnt2 = cnt // 2

    def _idx_start(g, rdst, cdst, sem):
        pltpu.async_copy(er_hbm.at[pl.ds((lo + g) * 128, 128)], rdst, sem)
        pltpu.async_copy(ec_hbm.at[pl.ds((lo + g) * 128, 128)], cdst, sem)

    def _idx_wait(g, rdst, cdst, sem):
        pltpu.make_async_copy(er_hbm.at[pl.ds((lo + g) * 128, 128)],
                              rdst, sem).wait()
        pltpu.make_async_copy(ec_hbm.at[pl.ds((lo + g) * 128, 128)],
                              cdst, sem).wait()

    pltpu.sync_copy(er_hbm.at[pl.ds(lo * 128, 128)], ridx0)
    pltpu.sync_copy(ec_hbm.at[pl.ds(lo * 128, 128)], cidx0)
    pltpu.async_copy(table.at[ridx0], bufa, sema)

    def body(k, carry):
        g = 2 * k
        pltpu.make_async_copy(table.at[ridx0], bufa, sema).wait()
        pltpu.async_copy(bufa, acc.at[cidx0], ssa, add=True)

        @pl.when(k > 0)
        def _():
            pltpu.make_async_copy(bufb, acc.at[ridx1], ssb).wait()
        _idx_start(g + 1, ridx1, cidx1, isem1)
        pltpu.async_copy(table.at[cidx0], bufb, semb)

        pltpu.make_async_copy(table.at[cidx0], bufb, semb).wait()
        pltpu.async_copy(bufb, acc.at[ridx0], ssb, add=True)
        pltpu.make_async_copy(bufa, acc.at[cidx0], ssa).wait()
        _idx_wait(g + 1, ridx1, cidx1, isem1)
        pltpu.async_copy(table.at[ridx1], bufa, sema)

        pltpu.make_async_copy(table.at[ridx1], bufa, sema).wait()
        pltpu.async_copy(bufa, acc.at[cidx1], ssa, add=True)
        pltpu.make_async_copy(bufb, acc.at[ridx0], ssb).wait()

        @pl.when(k < cnt2 - 1)
        def _():
            _idx_start(g + 2, ridx0, cidx0, isem0)
        pltpu.async_copy(table.at[cidx1], bufb, semb)

        pltpu.make_async_copy(table.at[cidx1], bufb, semb).wait()
        pltpu.async_copy(bufb, acc.at[ridx1], ssb, add=True)
        pltpu.make_async_copy(bufa, acc.at[cidx1], ssa).wait()

        @pl.when(k < cnt2 - 1)
        def _():
            _idx_wait(g + 2, ridx0, cidx0, isem0)
            pltpu.async_copy(table.at[ridx0], bufa, sema)

        return carry

    lax.fori_loop(0, cnt2, body, 0)
    pltpu.make_async_copy(bufb, acc.at[ridx1], ssb).wait()

    plsc.subcore_barrier()
    _copy_out(acc, bufa, out_hbm.at[c], s)


_sc_agg = pl.kernel(
    _sc_agg_body,
    out_type=jax.ShapeDtypeStruct((2, N, HALF), jnp.float32),
    mesh=_mesh,
    scratch_types=[
        pltpu.VMEM((128,), jnp.int32),
        pltpu.VMEM((128,), jnp.int32),
        pltpu.VMEM((128,), jnp.int32),
        pltpu.VMEM((128,), jnp.int32),
        pltpu.VMEM((128, HALF), jnp.float32),
        pltpu.VMEM((128, HALF), jnp.float32),
        pltpu.VMEM_SHARED((N, HALF), jnp.float32),
        pltpu.SemaphoreType.DMA,
        pltpu.SemaphoreType.DMA,
        pltpu.SemaphoreType.DMA,
        pltpu.SemaphoreType.DMA,
        pltpu.SemaphoreType.DMA,
        pltpu.SemaphoreType.DMA,
    ],
)


BM = 256
_GRID = (N + BM - 1) // BM


def _tmm_body(x_ref, w_ref, b_ref, out_ref):
    out_ref[...] = lax.dot_general(
        x_ref[...], w_ref[...], (((1,), (1,)), ((), ())),
        preferred_element_type=jnp.float32) + b_ref[...]


def _tscale_body(linu_ref, degp_ref, lin_ref, dis_ref):
    deg = degp_ref[0, :, 0:1] + degp_ref[1, :, 0:1] + 1.0
    dis = lax.rsqrt(deg)
    linp = dis * linu_ref[...]
    dis_ref[...] = dis
    lin_ref[0] = linp[:, :HALF]
    lin_ref[1] = linp[:, HALF:]


def _tmid_body(agg_ref, lp_ref, dis_ref, w_ref, b_ref, out_ref):
    dis = dis_ref[...]
    pre = jnp.concatenate(
        [agg_ref[0] + lp_ref[0], agg_ref[1] + lp_ref[1]], axis=1)
    h = jnp.maximum(dis * pre, 0.0)
    lin = lax.dot_general(h, w_ref[...], (((1,), (1,)), ((), ())),
                          preferred_element_type=jnp.float32) + b_ref[...]
    linp = dis * lin
    out_ref[0] = linp[:, :HALF]
    out_ref[1] = linp[:, HALF:]


def _tfin_body(agg_ref, lp_ref, dis_ref, pw_ref, bi_ref, h_ref, z_ref):
    pre = jnp.concatenate(
        [agg_ref[0] + lp_ref[0], agg_ref[1] + lp_ref[1]], axis=1)
    h = jnp.maximum(dis_ref[...] * pre, 0.0)
    h_ref[...] = h
    scores = lax.dot_general(h, pw_ref[...], (((1,), (0,)), ((), ())),
                             preferred_element_type=jnp.float32)
    smax = jnp.max(scores)
    e = jnp.exp(scores - smax)
    sm = e / jnp.sum(e)
    oh = (bi_ref[...] == lax.broadcasted_iota(jnp.int32, (N, B), 1)
          ).astype(jnp.float32)
    z_ref[...] = lax.dot_general(oh, h * sm, (((0,), (0,)), ((), ())),
                                 preferred_element_type=jnp.float32)


_tmm = pl.pallas_call(
    _tmm_body,
    grid=(_GRID,),
    in_specs=[
        pl.BlockSpec((BM, D), lambda i: (i, 0)),
        pl.BlockSpec((D, D), lambda i: (0, 0)),
        pl.BlockSpec((1, D), lambda i: (0, 0)),
    ],
    out_specs=pl.BlockSpec((BM, D), lambda i: (i, 0)),
    out_shape=jax.ShapeDtypeStruct((N, D), jnp.float32),
)

_tscale = pl.pallas_call(
    _tscale_body,
    grid=(_GRID,),
    in_specs=[
        pl.BlockSpec((BM, D), lambda i: (i, 0)),
        pl.BlockSpec((2, BM, HALF), lambda i: (0, i, 0)),
    ],
    out_specs=[
        pl.BlockSpec((2, BM, HALF), lambda i: (0, i, 0)),
        pl.BlockSpec((BM, 1), lambda i: (i, 0)),
    ],
    out_shape=[
        jax.ShapeDtypeStruct((2, N, HALF), jnp.float32),
        jax.ShapeDtypeStruct((N, 1), jnp.float32),
    ],
)

_tmid = pl.pallas_call(
    _tmid_body,
    grid=(_GRID,),
    in_specs=[
        pl.BlockSpec((2, BM, HALF), lambda i: (0, i, 0)),
        pl.BlockSpec((2, BM, HALF), lambda i: (0, i, 0)),
        pl.BlockSpec((BM, 1), lambda i: (i, 0)),
        pl.BlockSpec((D, D), lambda i: (0, 0)),
        pl.BlockSpec((1, D), lambda i: (0, 0)),
    ],
    out_specs=pl.BlockSpec((2, BM, HALF), lambda i: (0, i, 0)),
    out_shape=jax.ShapeDtypeStruct((2, N, HALF), jnp.float32),
)

_tfin = pl.pallas_call(
    _tfin_body,
    out_shape=[
        jax.ShapeDtypeStruct((N, D), jnp.float32),
        jax.ShapeDtypeStruct((B, D), jnp.float32),
    ],
)


def kernel(x, edge_index, batch_index, W0, b0, W1, b1, W2, b2, pool_w):
    er = edge_index[0]
    ec = edge_index[1]
    bi = batch_index.reshape(N, 1)

    degp = _sc_deg(edge_index.reshape(2 * E))
    linu = _tmm(x, W0, b0.reshape(1, D))
    lin0, dis = _tscale(linu, degp)
    agg0 = _sc_agg(lin0, er, ec)
    lin1 = _tmid(agg0, lin0, dis, W1, b1.reshape(1, D))
    agg1 = _sc_agg(lin1, er, ec)
    lin2 = _tmid(agg1, lin1, dis, W2, b2.reshape(1, D))
    agg2 = _sc_agg(lin2, er, ec)
    h, z = _tfin(agg2, lin2, dis, pool_w.reshape(D, 1), bi)
    return (h, z)


# speedup vs baseline: 15.9430x; 1.0003x over previous
import jax
import jax.numpy as jnp
from jax import lax
from jax.experimental import pallas as pl
from jax.experimental.pallas import tpu as pltpu
from jax.experimental.pallas import tpu_sc as plsc

N = 10000
E = 160000
NB = E // 128
D = 256
B = 64
HALF = 128
RA = 624

_mesh = plsc.VectorSubcoreMesh(core_axis_name="c", subcore_axis_name="s")


def _zero16():
    return jnp.zeros((16,), jnp.float32)


def _tile_batches(s):
    lo = s * 80
    cnt = jnp.where(s < 15, 80, 50)
    return lo, cnt


def _fill_const(buf, rows, vec):
    def body(t, carry):
        buf[t // 8, pl.ds((t % 8) * 16, 16)] = vec
        return carry
    lax.fori_loop(0, rows * 8, body, 0)


def _zero_acc(acc, zbuf, s):
    base = s * RA

    @pl.when(s < 15)
    def _():
        for k in range(4):
            pltpu.sync_copy(zbuf, acc.at[pl.ds(base + k * 128, 128)])
        pltpu.sync_copy(zbuf.at[pl.ds(0, RA - 512)],
                        acc.at[pl.ds(base + 512, RA - 512)])

    @pl.when(s == 15)
    def _():
        for k in range(5):
            pltpu.sync_copy(zbuf, acc.at[pl.ds(base + k * 128, 128)])


def _copy_out(acc, buf, out_half, s):
    base = s * RA

    @pl.when(s < 15)
    def _():
        for k in range(4):
            pltpu.sync_copy(acc.at[pl.ds(base + k * 128, 128)], buf)
            pltpu.sync_copy(buf, out_half.at[pl.ds(base + k * 128, 128)])
        pltpu.sync_copy(acc.at[pl.ds(base + 512, RA - 512)],
                        buf.at[pl.ds(0, RA - 512)])
        pltpu.sync_copy(buf.at[pl.ds(0, RA - 512)],
                        out_half.at[pl.ds(base + 512, RA - 512)])

    @pl.when(s == 15)
    def _():
        for k in range(5):
            pltpu.sync_copy(acc.at[pl.ds(base + k * 128, 128)], buf)
            pltpu.sync_copy(buf, out_half.at[pl.ds(base + k * 128, 128)])


def _sc_deg_body(ep_hbm, out_hbm, idx0, idx1, ones, zbuf, acc,
                 ssa, ssb, isem0, isem1):
    c = lax.axis_index("c")
    s = lax.axis_index("s")

    one16 = jnp.full((16,), 1.0, jnp.float32)
    _fill_const(ones, 128, one16)
    _fill_const(zbuf, 128, _zero16())
    _zero_acc(acc, zbuf, s)
    plsc.subcore_barrier()

    lo, cnt = _tile_batches(s)
    base = (c * 1250 + lo) * 128
    cnt2 = cnt // 2

    pltpu.sync_copy(ep_hbm.at[pl.ds(base, 128)], idx0)

    def body(k, carry):
        off = base + 2 * k * 128

        @pl.when(k > 0)
        def _():
            pltpu.make_async_copy(ep_hbm.at[pl.ds(off, 128)], idx0,
                                  isem0).wait()
        pltpu.async_copy(ones, acc.at[idx0], ssa, add=True)

        @pl.when(k > 0)
        def _():
            pltpu.make_async_copy(ones, acc.at[idx1], ssb).wait()
        pltpu.async_copy(ep_hbm.at[pl.ds(off + 128, 128)], idx1, isem1)

        pltpu.make_async_copy(ep_hbm.at[pl.ds(off + 128, 128)], idx1,
                              isem1).wait()
        pltpu.async_copy(ones, acc.at[idx1], ssb, add=True)
        pltpu.make_async_copy(ones, acc.at[idx0], ssa).wait()

        @pl.when(k < cnt2 - 1)
        def _():
            pltpu.async_copy(ep_hbm.at[pl.ds(off + 256, 128)], idx0, isem0)

        return carry

    lax.fori_loop(0, cnt2, body, 0)
    pltpu.make_async_copy(ones, acc.at[idx1], ssb).wait()

    plsc.subcore_barrier()
    _copy_out(acc, zbuf, out_hbm.at[c], s)


_sc_deg = pl.kernel(
    _sc_deg_body,
    out_type=jax.ShapeDtypeStruct((2, N, HALF), jnp.float32),
    mesh=_mesh,
    scratch_types=[
        pltpu.VMEM((128,), jnp.int32),
        pltpu.VMEM((128,), jnp.int32),
        pltpu.VMEM((128, HALF), jnp.float32),
        pltpu.VMEM((128, HALF), jnp.float32),
        pltpu.VMEM_SHARED((N, HALF), jnp.float32),
        pltpu.SemaphoreType.DMA,
        pltpu.SemaphoreType.DMA,
        pltpu.SemaphoreType.DMA,
        pltpu.SemaphoreType.DMA,
    ],
)


def _sc_agg_body(lin_hbm, er_hbm, ec_hbm, out_hbm,
                 ridx0, ridx1, cidx0, cidx1, bufa, bufb, acc,
                 sema, semb, isem0, isem1, ssa, ssb):
    c = lax.axis_index("c")
    s = lax.axis_index("s")

    _fill_const(bufa, 128, _zero16())
    _zero_acc(acc, bufa, s)
    plsc.subcore_barrier()

    lo, cnt = _tile_batches(s)
    table = lin_hbm.at[c]
    cnt2 = cnt // 2

    def _idx_start(g, rdst, cdst, sem):
        pltpu.async_copy(er_hbm.at[pl.ds((lo + g) * 128, 128)], rdst, sem)
        pltpu.async_copy(ec_hbm.at[pl.ds((lo + g) * 128, 128)], cdst, sem)

    def _idx_wait(g, rdst, cdst, sem):
        pltpu.make_async_copy(er_hbm.at[pl.ds((lo + g) * 128, 128)],
                              rdst, sem).wait()
        pltpu.make_async_copy(ec_hbm.at[pl.ds((lo + g) * 128, 128)],
                              cdst, sem).wait()

    pltpu.sync_copy(er_hbm.at[pl.ds(lo * 128, 128)], ridx0)
    pltpu.sync_copy(ec_hbm.at[pl.ds(lo * 128, 128)], cidx0)
    pltpu.async_copy(table.at[ridx0], bufa, sema)

    def body(k, carry):
        g = 2 * k
        pltpu.make_async_copy(table.at[ridx0], bufa, sema).wait()
        pltpu.async_copy(bufa, acc.at[cidx0], ssa, add=True)

        @pl.when(k > 0)
        def _():
            pltpu.make_async_copy(bufb, acc.at[ridx1], ssb).wait()
        _idx_start(g + 1, ridx1, cidx1, isem1)
        pltpu.async_copy(table.at[cidx0], bufb, semb)

        pltpu.make_async_copy(table.at[cidx0], bufb, semb).wait()
        pltpu.async_copy(bufb, acc.at[ridx0], ssb, add=True)
        pltpu.make_async_copy(bufa, acc.at[cidx0], ssa).wait()
        _idx_wait(g + 1, ridx1, cidx1, isem1)
        pltpu.async_copy(table.at[ridx1], bufa, sema)

        pltpu.make_async_copy(table.at[ridx1], bufa, sema).wait()
        pltpu.async_copy(bufa, acc.at[cidx1], ssa, add=True)
        pltpu.make_async_copy(bufb, acc.at[ridx0], ssb).wait()

        @pl.when(k < cnt2 - 1)
        def _():
            _idx_start(g + 2, ridx0, cidx0, isem0)
        pltpu.async_copy(table.at[cidx1], bufb, semb)

        pltpu.make_async_copy(table.at[cidx1], bufb, semb).wait()
        pltpu.async_copy(bufb, acc.at[ridx1], ssb, add=True)
        pltpu.make_async_copy(bufa, acc.at[cidx1], ssa).wait()

        @pl.when(k < cnt2 - 1)
        def _():
            _idx_wait(g + 2, ridx0, cidx0, isem0)
            pltpu.async_copy(table.at[ridx0], bufa, sema)

        return carry

    lax.fori_loop(0, cnt2, body, 0)
    pltpu.make_async_copy(bufb, acc.at[ridx1], ssb).wait()

    plsc.subcore_barrier()
    _copy_out(acc, bufa, out_hbm.at[c], s)


_sc_agg = pl.kernel(
    _sc_agg_body,
    out_type=jax.ShapeDtypeStruct((2, N, HALF), jnp.float32),
    mesh=_mesh,
    scratch_types=[
        pltpu.VMEM((128,), jnp.int32),
        pltpu.VMEM((128,), jnp.int32),
        pltpu.VMEM((128,), jnp.int32),
        pltpu.VMEM((128,), jnp.int32),
        pltpu.VMEM((128, HALF), jnp.float32),
        pltpu.VMEM((128, HALF), jnp.float32),
        pltpu.VMEM_SHARED((N, HALF), jnp.float32),
        pltpu.SemaphoreType.DMA,
        pltpu.SemaphoreType.DMA,
        pltpu.SemaphoreType.DMA,
        pltpu.SemaphoreType.DMA,
        pltpu.SemaphoreType.DMA,
        pltpu.SemaphoreType.DMA,
    ],
)


BM = 256
_GRID = (N + BM - 1) // BM


def _tmm_body(x_ref, w_ref, b_ref, out_ref):
    out_ref[...] = lax.dot_general(
        x_ref[...], w_ref[...], (((1,), (1,)), ((), ())),
        preferred_element_type=jnp.float32) + b_ref[...]


def _tscale_body(linu_ref, degp_ref, lin_ref, dis_ref):
    deg = degp_ref[0, :, 0:1] + degp_ref[1, :, 0:1] + 1.0
    dis = lax.rsqrt(deg)
    linp = dis * linu_ref[...]
    dis_ref[...] = dis
    lin_ref[0] = linp[:, :HALF]
    lin_ref[1] = linp[:, HALF:]


def _tmid_body(agg_ref, lp_ref, dis_ref, w_ref, b_ref, out_ref):
    dis = dis_ref[...]
    pre = jnp.concatenate(
        [agg_ref[0] + lp_ref[0], agg_ref[1] + lp_ref[1]], axis=1)
    h = jnp.maximum(dis * pre, 0.0)
    lin = lax.dot_general(h, w_ref[...], (((1,), (1,)), ((), ())),
                          preferred_element_type=jnp.float32) + b_ref[...]
    linp = dis * lin
    out_ref[0] = linp[:, :HALF]
    out_ref[1] = linp[:, HALF:]


def _tfin_body(agg_ref, lp_ref, dis_ref, pw_ref, bi_ref, h_ref, z_ref):
    pre = jnp.concatenate(
        [agg_ref[0] + lp_ref[0], agg_ref[1] + lp_ref[1]], axis=1)
    h = jnp.maximum(dis_ref[...] * pre, 0.0)
    h_ref[...] = h
    scores = lax.dot_general(h, pw_ref[...], (((1,), (0,)), ((), ())),
                             preferred_element_type=jnp.float32)
    smax = jnp.max(scores)
    e = jnp.exp(scores - smax)
    sm = e / jnp.sum(e)
    oh = (bi_ref[...] == lax.broadcasted_iota(jnp.int32, (N, B), 1)
          ).astype(jnp.float32)
    z_ref[...] = lax.dot_general(oh, h * sm, (((0,), (0,)), ((), ())),
                                 preferred_element_type=jnp.float32)


_tmm = pl.pallas_call(
    _tmm_body,
    grid=(_GRID,),
    in_specs=[
        pl.BlockSpec((BM, D), lambda i: (i, 0)),
        pl.BlockSpec((D, D), lambda i: (0, 0)),
        pl.BlockSpec((1, D), lambda i: (0, 0)),
    ],
    out_specs=pl.BlockSpec((BM, D), lambda i: (i, 0)),
    out_shape=jax.ShapeDtypeStruct((N, D), jnp.float32),
)

_tscale = pl.pallas_call(
    _tscale_body,
    grid=(_GRID,),
    in_specs=[
        pl.BlockSpec((BM, D), lambda i: (i, 0)),
        pl.BlockSpec((2, BM, HALF), lambda i: (0, i, 0)),
    ],
    out_specs=[
        pl.BlockSpec((2, BM, HALF), lambda i: (0, i, 0)),
        pl.BlockSpec((BM, 1), lambda i: (i, 0)),
    ],
    out_shape=[
        jax.ShapeDtypeStruct((2, N, HALF), jnp.float32),
        jax.ShapeDtypeStruct((N, 1), jnp.float32),
    ],
)

_tmid = pl.pallas_call(
    _tmid_body,
    grid=(_GRID,),
    in_specs=[
        pl.BlockSpec((2, BM, HALF), lambda i: (0, i, 0)),
        pl.BlockSpec((2, BM, HALF), lambda i: (0, i, 0)),
        pl.BlockSpec((BM, 1), lambda i: (i, 0)),
        pl.BlockSpec((D, D), lambda i: (0, 0)),
        pl.BlockSpec((1, D), lambda i: (0, 0)),
    ],
    out_specs=pl.BlockSpec((2, BM, HALF), lambda i: (0, i, 0)),
    out_shape=jax.ShapeDtypeStruct((2, N, HALF), jnp.float32),
)

_tfin = pl.pallas_call(
    _tfin_body,
    out_shape=[
        jax.ShapeDtypeStruct((N, D), jnp.float32),
        jax.ShapeDtypeStruct((B, D), jnp.float32),
    ],
)


def kernel(x, edge_index, batch_index, W0, b0, W1, b1, W2, b2, pool_w):
    er = edge_index[0]
    ec = edge_index[1]
    bi = batch_index.reshape(N, 1)

    degp = _sc_deg(edge_index.reshape(2 * E))
    linu = _tmm(x, W0, b0.reshape(1, D))
    lin0, dis = _tscale(linu, degp)
    agg0 = _sc_agg(lin0, er, ec)
    lin1 = _tmid(agg0, lin0, dis, W1, b1.reshape(1, D))
    agg1 = _sc_agg(lin1, er, ec)
    lin2 = _tmid(agg1, lin1, dis, W2, b2.reshape(1, D))
    agg2 = _sc_agg(lin2, er, ec)
    h, z = _tfin(agg2, lin2, dis, pool_w.reshape(D, 1), bi)
    return (h, z)
